# baseline (speedup 1.0000x reference)
import functools

import jax
import jax.numpy as jnp
from jax import lax
from jax.experimental import pallas as pl
from jax.experimental.pallas import tpu as pltpu
from jax.experimental.pallas import tpu_sc as plsc

_N_AGENTS = 6
_N_ACTIONS = 10
_BATCH = 16384
_NC = 2
_NS = 16
_L = 16
_NW = _NC * _NS
_BPW = _BATCH // _NW
_EPW = _BPW * _N_AGENTS
_GCHUNK = 128
_NCHUNK = _EPW // _GCHUNK


def _build():
  mesh = plsc.VectorSubcoreMesh(core_axis_name="c", subcore_axis_name="s")

  @functools.partial(
      pl.kernel,
      mesh=mesh,
      out_type=jax.ShapeDtypeStruct((_BATCH * _N_AGENTS // _GCHUNK, _GCHUNK),
                                    jnp.float32),
      compiler_params=pltpu.CompilerParams(
          use_tc_tiling_on_sc=False, needs_layout_passes=False),
      scratch_types=[
          pltpu.VMEM((_EPW,), jnp.int32),
          pltpu.VMEM((_BPW,), jnp.int32),
          pltpu.VMEM((_NCHUNK, _GCHUNK), jnp.int32),
          pltpu.VMEM((_NCHUNK, _GCHUNK), jnp.float32),
          pltpu.SemaphoreType.DMA,
      ],
  )
  def _k(a_hbm, u_hbm, out_hbm, a_v, idx_v, eidx_v, rows_v, sem):
    wid = lax.axis_index("s") * _NC + lax.axis_index("c")
    base = wid * _BPW
    pltpu.sync_copy(a_hbm.at[pl.ds(base * _N_AGENTS, _EPW)], a_v)

    def jbody(g, carry):
      flat = (g * _L + lax.iota(jnp.int32, _L)) * _N_AGENTS
      acc = jnp.zeros((_L,), jnp.int32)
      scale = 1
      for i in range(_N_AGENTS):
        acc = acc + plsc.load_gather(a_v, [flat + i]) * scale
        scale *= _N_ACTIONS
      idx_v[pl.ds(g * _L, _L)] = acc
      return carry

    lax.fori_loop(0, _BPW // _L, jbody, 0)

    lanes = lax.iota(jnp.int32, _L)

    for j in range(_NCHUNK):
      def ebody(g, carry, j=j):
        k = j * _GCHUNK + g * _L + lanes
        q = k // _N_AGENTS
        c = k - q * _N_AGENTS
        e = plsc.load_gather(idx_v, [q]) * _N_AGENTS + c
        eidx_v[j, pl.ds(g * _L, _L)] = e
        return carry

      lax.fori_loop(0, _GCHUNK // _L, ebody, 0)

    copies = [
        pltpu.async_copy(u_hbm.at[eidx_v.at[j]], rows_v.at[j], sem)
        for j in range(_NCHUNK)
    ]
    for c in copies:
      c.wait()
    pltpu.sync_copy(rows_v, out_hbm.at[pl.ds(wid * _NCHUNK, _NCHUNK)])

  return _k


_sc_gather = _build()


def kernel(a_joint, U):
  out = _sc_gather(a_joint.reshape(-1), U.reshape(-1))
  return out.reshape(_BATCH, _N_AGENTS)


# speedup vs baseline: 4.0167x; 4.0167x over previous
import functools

import jax
import jax.numpy as jnp
from jax import lax
from jax.experimental import pallas as pl
from jax.experimental.pallas import tpu as pltpu
from jax.experimental.pallas import tpu_sc as plsc

_N_AGENTS = 6
_N_ACTIONS = 10
_BATCH = 16384
_NC = 2
_NS = 16
_L = 16
_NW = _NC * _NS
_BPW = _BATCH // _NW
_GCHUNK = 128
_NCHUNK = _BPW // _GCHUNK


def _build():
  mesh = plsc.VectorSubcoreMesh(core_axis_name="c", subcore_axis_name="s")

  @functools.partial(
      pl.kernel,
      mesh=mesh,
      out_type=tuple(
          jax.ShapeDtypeStruct((_BATCH,), jnp.float32)
          for _ in range(_N_AGENTS)
      ),
      compiler_params=pltpu.CompilerParams(
          use_tc_tiling_on_sc=False, needs_layout_passes=False),
      scratch_types=[
          pltpu.VMEM((_N_AGENTS, _BPW), jnp.int32),
          pltpu.VMEM((_NCHUNK, _GCHUNK), jnp.int32),
          pltpu.VMEM((_N_AGENTS * _NCHUNK, _GCHUNK), jnp.float32),
          pltpu.SemaphoreType.DMA,
      ],
  )
  def _k(a0, a1, a2, a3, a4, a5, u0, u1, u2, u3, u4, u5,
         o0, o1, o2, o3, o4, o5, a_v, idx_v, rows_v, sem):
    a_cols = (a0, a1, a2, a3, a4, a5)
    u_cols = (u0, u1, u2, u3, u4, u5)
    o_cols = (o0, o1, o2, o3, o4, o5)
    wid = lax.axis_index("s") * _NC + lax.axis_index("c")
    base = wid * _BPW
    for i in range(_N_AGENTS):
      pltpu.sync_copy(a_cols[i].at[pl.ds(base, _BPW)], a_v.at[i])

    for j in range(_NCHUNK):
      def jbody(g, carry, j=j):
        off = j * _GCHUNK + g * _L
        acc = jnp.zeros((_L,), jnp.int32)
        scale = 1
        for i in range(_N_AGENTS):
          acc = acc + a_v[i, pl.ds(off, _L)] * scale
          scale *= _N_ACTIONS
        idx_v[j, pl.ds(g * _L, _L)] = acc
        return carry

      lax.fori_loop(0, _GCHUNK // _L, jbody, 0)

    copies = [
        pltpu.async_copy(
            u_cols[c].at[idx_v.at[j]],
            rows_v.at[c * _NCHUNK + j],
            sem,
        )
        for c in range(_N_AGENTS)
        for j in range(_NCHUNK)
    ]
    for cp in copies:
      cp.wait()
    for c in range(_N_AGENTS):
      for j in range(_NCHUNK):
        pltpu.sync_copy(
            rows_v.at[c * _NCHUNK + j],
            o_cols[c].at[pl.ds(base + j * _GCHUNK, _GCHUNK)],
        )

  return _k


_sc_gather = _build()


def kernel(a_joint, U):
  a_cols = tuple(a_joint[:, i] for i in range(_N_AGENTS))
  u_cols = tuple(U[:, i] for i in range(_N_AGENTS))
  outs = _sc_gather(*a_cols, *u_cols)
  return jnp.stack(outs, axis=1)


# speedup vs baseline: 5.2497x; 1.3070x over previous
import functools

import jax
import jax.numpy as jnp
from jax import lax
from jax.experimental import pallas as pl
from jax.experimental.pallas import tpu as pltpu
from jax.experimental.pallas import tpu_sc as plsc

_N_AGENTS = 6
_N_ACTIONS = 10
_BATCH = 16384
_NC = 2
_NS = 16
_L = 16
_NW = _NC * _NS
_BPW = _BATCH // _NW
_GCHUNK = 128
_NCHUNK = _BPW // _GCHUNK


def _build():
  mesh = plsc.VectorSubcoreMesh(core_axis_name="c", subcore_axis_name="s")

  @functools.partial(
      pl.kernel,
      mesh=mesh,
      out_type=tuple(
          jax.ShapeDtypeStruct((_BATCH,), jnp.float32)
          for _ in range(_N_AGENTS)
      ),
      compiler_params=pltpu.CompilerParams(
          use_tc_tiling_on_sc=False, needs_layout_passes=False),
      scratch_types=[
          pltpu.VMEM((_N_AGENTS, _BPW), jnp.int32),
          pltpu.VMEM((_NCHUNK, _GCHUNK), jnp.int32),
          pltpu.VMEM((_N_AGENTS * _NCHUNK, _GCHUNK), jnp.float32),
          pltpu.SemaphoreType.DMA,
      ],
  )
  def _k(a0, a1, a2, a3, a4, a5, u0, u1, u2, u3, u4, u5,
         o0, o1, o2, o3, o4, o5, a_v, idx_v, rows_v, sem):
    a_cols = (a0, a1, a2, a3, a4, a5)
    u_cols = (u0, u1, u2, u3, u4, u5)
    o_cols = (o0, o1, o2, o3, o4, o5)
    wid = lax.axis_index("s") * _NC + lax.axis_index("c")
    base = wid * _BPW
    for i in range(_N_AGENTS):
      pltpu.sync_copy(a_cols[i].at[pl.ds(base, _BPW)], a_v.at[i])

    for j in range(_NCHUNK):
      def jbody(g, carry, j=j):
        off = j * _GCHUNK + g * _L
        acc = jnp.zeros((_L,), jnp.int32)
        scale = 1
        for i in range(_N_AGENTS):
          acc = acc + a_v[i, pl.ds(off, _L)] * scale
          scale *= _N_ACTIONS
        idx_v[j, pl.ds(g * _L, _L)] = acc
        return carry

      lax.fori_loop(0, _GCHUNK // _L, jbody, 0)

    copies = [
        pltpu.async_copy(
            u_cols[c].at[idx_v.at[j]],
            rows_v.at[c * _NCHUNK + j],
            sem,
        )
        for c in range(_N_AGENTS)
        for j in range(_NCHUNK)
    ]
    for cp in copies:
      cp.wait()
    for c in range(_N_AGENTS):
      for j in range(_NCHUNK):
        pltpu.sync_copy(
            rows_v.at[c * _NCHUNK + j],
            o_cols[c].at[pl.ds(base + j * _GCHUNK, _GCHUNK)],
        )

  return _k


_sc_gather = _build()

_K = 1000000
_BN = 8192


def _extract_body(ut_ref, *out_refs):
  for c in range(_N_AGENTS):
    out_refs[c][...] = ut_ref[c, :]


def _build_extract():
  return pl.pallas_call(
      _extract_body,
      grid=(-(-_K // _BN),),
      in_specs=[pl.BlockSpec((_N_AGENTS, _BN), lambda k: (0, k))],
      out_specs=[
          pl.BlockSpec((_BN,), lambda k: (k,)) for _ in range(_N_AGENTS)
      ],
      out_shape=tuple(
          jax.ShapeDtypeStruct((_K,), jnp.float32) for _ in range(_N_AGENTS)
      ),
  )


_tc_extract = _build_extract()


def kernel(a_joint, U):
  a_cols = tuple(a_joint[:, i] for i in range(_N_AGENTS))
  u_cols = _tc_extract(U.T)
  outs = _sc_gather(*a_cols, *u_cols)
  return jnp.stack(outs, axis=1)


# speedup vs baseline: 5.2714x; 1.0041x over previous
import functools

import jax
import jax.numpy as jnp
from jax import lax
from jax.experimental import pallas as pl
from jax.experimental.pallas import tpu as pltpu
from jax.experimental.pallas import tpu_sc as plsc

_N_AGENTS = 6
_N_ACTIONS = 10
_BATCH = 16384
_K = 1000000
_NC = 2
_NS = 16
_L = 16
_NW = _NC * _NS
_BPW = _BATCH // _NW
_GCHUNK = 128
_NCHUNK = _BPW // _GCHUNK

_BN = 8192
_NB = -(-_K // _BN)
_WWORDS = _NB * _N_AGENTS * _BN


def _pack_body(ut_ref, w_ref):
  w_ref[...] = ut_ref[...].reshape(1, _N_AGENTS, _BN // 128, 128)


def _build_pack():
  return pl.pallas_call(
      _pack_body,
      grid=(_NB,),
      in_specs=[pl.BlockSpec((_N_AGENTS, _BN), lambda k: (0, k))],
      out_specs=pl.BlockSpec(
          (1, _N_AGENTS, _BN // 128, 128), lambda k: (k, 0, 0, 0)),
      out_shape=jax.ShapeDtypeStruct(
          (_NB, _N_AGENTS, _BN // 128, 128), jnp.float32),
  )


_tc_pack = _build_pack()


def _build_gather():
  mesh = plsc.VectorSubcoreMesh(core_axis_name="c", subcore_axis_name="s")

  @functools.partial(
      pl.kernel,
      mesh=mesh,
      out_type=tuple(
          jax.ShapeDtypeStruct((_BATCH,), jnp.float32)
          for _ in range(_N_AGENTS)
      ),
      compiler_params=pltpu.CompilerParams(
          use_tc_tiling_on_sc=False, needs_layout_passes=False),
      scratch_types=[
          pltpu.VMEM((_N_AGENTS, _BPW), jnp.int32),
          pltpu.VMEM((_N_AGENTS * _NCHUNK, _GCHUNK), jnp.int32),
          pltpu.VMEM((_N_AGENTS * _NCHUNK, _GCHUNK), jnp.float32),
          pltpu.SemaphoreType.DMA,
      ],
  )
  def _k(a0, a1, a2, a3, a4, a5, w_hbm,
         o0, o1, o2, o3, o4, o5, a_v, eidx_v, rows_v, sem):
    a_cols = (a0, a1, a2, a3, a4, a5)
    o_cols = (o0, o1, o2, o3, o4, o5)
    wid = lax.axis_index("s") * _NC + lax.axis_index("c")
    base = wid * _BPW
    for i in range(_N_AGENTS):
      pltpu.sync_copy(a_cols[i].at[pl.ds(base, _BPW)], a_v.at[i])

    for j in range(_NCHUNK):
      def jbody(g, carry, j=j):
        off = j * _GCHUNK + g * _L
        acc = jnp.zeros((_L,), jnp.int32)
        scale = 1
        for i in range(_N_AGENTS):
          acc = acc + a_v[i, pl.ds(off, _L)] * scale
          scale *= _N_ACTIONS
        addr0 = (acc >> 13) * (_N_AGENTS * _BN) + (acc & (_BN - 1))
        for c in range(_N_AGENTS):
          eidx_v[c * _NCHUNK + j, pl.ds(g * _L, _L)] = addr0 + c * _BN
        return carry

      lax.fori_loop(0, _GCHUNK // _L, jbody, 0)

    copies = [
        pltpu.async_copy(w_hbm.at[eidx_v.at[n]], rows_v.at[n], sem)
        for n in range(_N_AGENTS * _NCHUNK)
    ]
    for cp in copies:
      cp.wait()
    for c in range(_N_AGENTS):
      for j in range(_NCHUNK):
        pltpu.sync_copy(
            rows_v.at[c * _NCHUNK + j],
            o_cols[c].at[pl.ds(base + j * _GCHUNK, _GCHUNK)],
        )

  return _k


_sc_gather = _build_gather()


def kernel(a_joint, U):
  a_cols = tuple(a_joint[:, i] for i in range(_N_AGENTS))
  w_flat = _tc_pack(U.T).reshape(_WWORDS)
  outs = _sc_gather(*a_cols, w_flat)
  return jnp.stack(outs, axis=1)


# speedup vs baseline: 8.8669x; 1.6821x over previous
import functools

import jax
import jax.numpy as jnp
from jax import lax
from jax.experimental import pallas as pl
from jax.experimental.pallas import tpu as pltpu
from jax.experimental.pallas import tpu_sc as plsc

_N_AGENTS = 6
_N_ACTIONS = 10
_BATCH = 16384
_K = 1000000
_NC = 2
_NS = 16
_L = 16
_NW = _NC * _NS
_BPW = _BATCH // _NW
_GCHUNK = 128
_NCHUNK = _BPW // _GCHUNK

_BN = 32768
_NB = -(-_K // _BN)
_BSHIFT = _BN.bit_length() - 1
_WWORDS = _NB * _N_AGENTS * _BN


def _pack_body(ut_ref, w_ref):
  w_ref[...] = ut_ref[...].reshape(1, _N_AGENTS, _BN // 128, 128)


def _build_pack():
  return pl.pallas_call(
      _pack_body,
      grid=(_NB,),
      in_specs=[pl.BlockSpec((_N_AGENTS, _BN), lambda k: (0, k))],
      out_specs=pl.BlockSpec(
          (1, _N_AGENTS, _BN // 128, 128), lambda k: (k, 0, 0, 0)),
      out_shape=jax.ShapeDtypeStruct(
          (_NB, _N_AGENTS, _BN // 128, 128), jnp.float32),
  )


_tc_pack = _build_pack()


def _build_gather():
  mesh = plsc.VectorSubcoreMesh(core_axis_name="c", subcore_axis_name="s")

  @functools.partial(
      pl.kernel,
      mesh=mesh,
      out_type=tuple(
          jax.ShapeDtypeStruct((_BATCH,), jnp.float32)
          for _ in range(_N_AGENTS)
      ),
      compiler_params=pltpu.CompilerParams(
          use_tc_tiling_on_sc=False, needs_layout_passes=False),
      scratch_types=[
          pltpu.VMEM((_N_AGENTS, _BPW), jnp.int32),
          pltpu.VMEM((_N_AGENTS * _NCHUNK, _GCHUNK), jnp.int32),
          pltpu.VMEM((_N_AGENTS * _NCHUNK, _GCHUNK), jnp.float32),
          pltpu.SemaphoreType.DMA,
      ],
  )
  def _k(a0, a1, a2, a3, a4, a5, w_hbm,
         o0, o1, o2, o3, o4, o5, a_v, eidx_v, rows_v, sem):
    a_cols = (a0, a1, a2, a3, a4, a5)
    o_cols = (o0, o1, o2, o3, o4, o5)
    wid = lax.axis_index("s") * _NC + lax.axis_index("c")
    base = wid * _BPW
    for i in range(_N_AGENTS):
      pltpu.sync_copy(a_cols[i].at[pl.ds(base, _BPW)], a_v.at[i])

    for j in range(_NCHUNK):
      def jbody(g, carry, j=j):
        off = j * _GCHUNK + g * _L
        acc = jnp.zeros((_L,), jnp.int32)
        scale = 1
        for i in range(_N_AGENTS):
          acc = acc + a_v[i, pl.ds(off, _L)] * scale
          scale *= _N_ACTIONS
        addr0 = (acc >> _BSHIFT) * (_N_AGENTS * _BN) + (acc & (_BN - 1))
        for c in range(_N_AGENTS):
          eidx_v[c * _NCHUNK + j, pl.ds(g * _L, _L)] = addr0 + c * _BN
        return carry

      lax.fori_loop(0, _GCHUNK // _L, jbody, 0)

    copies = [
        pltpu.async_copy(w_hbm.at[eidx_v.at[n]], rows_v.at[n], sem)
        for n in range(_N_AGENTS * _NCHUNK)
    ]
    for cp in copies:
      cp.wait()
    for c in range(_N_AGENTS):
      for j in range(_NCHUNK):
        pltpu.sync_copy(
            rows_v.at[c * _NCHUNK + j],
            o_cols[c].at[pl.ds(base + j * _GCHUNK, _GCHUNK)],
        )

  return _k


_sc_gather = _build_gather()


def kernel(a_joint, U):
  a_cols = tuple(a_joint[:, i] for i in range(_N_AGENTS))
  w_flat = _tc_pack(U.T).reshape(_WWORDS)
  outs = _sc_gather(*a_cols, w_flat)
  return jnp.stack(outs, axis=1)


# speedup vs baseline: 10.7192x; 1.2089x over previous
import functools

import jax
import jax.numpy as jnp
from jax import lax
from jax.experimental import pallas as pl
from jax.experimental.pallas import tpu as pltpu
from jax.experimental.pallas import tpu_sc as plsc

_N_AGENTS = 6
_N_ACTIONS = 10
_BATCH = 16384
_K = 1000000
_NC = 2
_NS = 16
_L = 16
_NW = _NC * _NS
_BPW = _BATCH // _NW
_GCHUNK = 128
_NCHUNK = _BPW // _GCHUNK

_BN = 131072
_NB = -(-_K // _BN)
_BSHIFT = _BN.bit_length() - 1
_WWORDS = _NB * _N_AGENTS * _BN


def _pack_body(ut_ref, w_ref):
  w_ref[...] = ut_ref[...].reshape(1, _N_AGENTS, _BN // 128, 128)


def _build_pack():
  return pl.pallas_call(
      _pack_body,
      grid=(_NB,),
      in_specs=[pl.BlockSpec((_N_AGENTS, _BN), lambda k: (0, k))],
      out_specs=pl.BlockSpec(
          (1, _N_AGENTS, _BN // 128, 128), lambda k: (k, 0, 0, 0)),
      out_shape=jax.ShapeDtypeStruct(
          (_NB, _N_AGENTS, _BN // 128, 128), jnp.float32),
  )


_tc_pack = _build_pack()


def _build_gather():
  mesh = plsc.VectorSubcoreMesh(core_axis_name="c", subcore_axis_name="s")

  @functools.partial(
      pl.kernel,
      mesh=mesh,
      out_type=tuple(
          jax.ShapeDtypeStruct((_BATCH,), jnp.float32)
          for _ in range(_N_AGENTS)
      ),
      compiler_params=pltpu.CompilerParams(
          use_tc_tiling_on_sc=False, needs_layout_passes=False),
      scratch_types=[
          pltpu.VMEM((_N_AGENTS, _BPW), jnp.int32),
          pltpu.VMEM((_N_AGENTS * _NCHUNK, _GCHUNK), jnp.int32),
          pltpu.VMEM((_N_AGENTS * _NCHUNK, _GCHUNK), jnp.float32),
          pltpu.SemaphoreType.DMA,
      ],
  )
  def _k(a0, a1, a2, a3, a4, a5, w_hbm,
         o0, o1, o2, o3, o4, o5, a_v, eidx_v, rows_v, sem):
    a_cols = (a0, a1, a2, a3, a4, a5)
    o_cols = (o0, o1, o2, o3, o4, o5)
    wid = lax.axis_index("s") * _NC + lax.axis_index("c")
    base = wid * _BPW
    for i in range(_N_AGENTS):
      pltpu.sync_copy(a_cols[i].at[pl.ds(base, _BPW)], a_v.at[i])

    for j in range(_NCHUNK):
      def jbody(g, carry, j=j):
        off = j * _GCHUNK + g * _L
        acc = jnp.zeros((_L,), jnp.int32)
        scale = 1
        for i in range(_N_AGENTS):
          acc = acc + a_v[i, pl.ds(off, _L)] * scale
          scale *= _N_ACTIONS
        addr0 = (acc >> _BSHIFT) * (_N_AGENTS * _BN) + (acc & (_BN - 1))
        for c in range(_N_AGENTS):
          eidx_v[c * _NCHUNK + j, pl.ds(g * _L, _L)] = addr0 + c * _BN
        return carry

      lax.fori_loop(0, _GCHUNK // _L, jbody, 0)

    copies = [
        pltpu.async_copy(w_hbm.at[eidx_v.at[n]], rows_v.at[n], sem)
        for n in range(_N_AGENTS * _NCHUNK)
    ]
    for cp in copies:
      cp.wait()
    for c in range(_N_AGENTS):
      for j in range(_NCHUNK):
        pltpu.sync_copy(
            rows_v.at[c * _NCHUNK + j],
            o_cols[c].at[pl.ds(base + j * _GCHUNK, _GCHUNK)],
        )

  return _k


_sc_gather = _build_gather()


def kernel(a_joint, U):
  a_cols = tuple(a_joint[:, i] for i in range(_N_AGENTS))
  w_flat = _tc_pack(U.T).reshape(_WWORDS)
  outs = _sc_gather(*a_cols, w_flat)
  return jnp.stack(outs, axis=1)


# speedup vs baseline: 10.8865x; 1.0156x over previous
import functools

import jax
import jax.numpy as jnp
from jax import lax
from jax.experimental import pallas as pl
from jax.experimental.pallas import tpu as pltpu
from jax.experimental.pallas import tpu_sc as plsc

_N_AGENTS = 6
_N_ACTIONS = 10
_BATCH = 16384
_K = 1000000
_NC = 2
_NS = 16
_L = 16
_NW = _NC * _NS
_BPW = _BATCH // _NW
_GCHUNK = 128
_NCHUNK = _BPW // _GCHUNK

_BN = 262144
_NB = -(-_K // _BN)
_BSHIFT = _BN.bit_length() - 1
_WWORDS = _NB * _N_AGENTS * _BN


def _pack_body(ut_ref, w_ref):
  w_ref[...] = ut_ref[...].reshape(1, _N_AGENTS, _BN // 128, 128)


def _build_pack():
  return pl.pallas_call(
      _pack_body,
      grid=(_NB,),
      in_specs=[pl.BlockSpec((_N_AGENTS, _BN), lambda k: (0, k))],
      out_specs=pl.BlockSpec(
          (1, _N_AGENTS, _BN // 128, 128), lambda k: (k, 0, 0, 0)),
      out_shape=jax.ShapeDtypeStruct(
          (_NB, _N_AGENTS, _BN // 128, 128), jnp.float32),
  )


_tc_pack = _build_pack()


def _build_gather():
  mesh = plsc.VectorSubcoreMesh(core_axis_name="c", subcore_axis_name="s")

  @functools.partial(
      pl.kernel,
      mesh=mesh,
      out_type=tuple(
          jax.ShapeDtypeStruct((_BATCH,), jnp.float32)
          for _ in range(_N_AGENTS)
      ),
      compiler_params=pltpu.CompilerParams(
          use_tc_tiling_on_sc=False, needs_layout_passes=False),
      scratch_types=[
          pltpu.VMEM((_N_AGENTS, _BPW), jnp.int32),
          pltpu.VMEM((_N_AGENTS * _NCHUNK, _GCHUNK), jnp.int32),
          pltpu.VMEM((_N_AGENTS * _NCHUNK, _GCHUNK), jnp.float32),
          pltpu.SemaphoreType.DMA,
      ],
  )
  def _k(a0, a1, a2, a3, a4, a5, w_hbm,
         o0, o1, o2, o3, o4, o5, a_v, eidx_v, rows_v, sem):
    a_cols = (a0, a1, a2, a3, a4, a5)
    o_cols = (o0, o1, o2, o3, o4, o5)
    wid = lax.axis_index("s") * _NC + lax.axis_index("c")
    base = wid * _BPW
    for i in range(_N_AGENTS):
      pltpu.sync_copy(a_cols[i].at[pl.ds(base, _BPW)], a_v.at[i])

    for j in range(_NCHUNK):
      def jbody(g, carry, j=j):
        off = j * _GCHUNK + g * _L
        acc = jnp.zeros((_L,), jnp.int32)
        scale = 1
        for i in range(_N_AGENTS):
          acc = acc + a_v[i, pl.ds(off, _L)] * scale
          scale *= _N_ACTIONS
        addr0 = (acc >> _BSHIFT) * (_N_AGENTS * _BN) + (acc & (_BN - 1))
        for c in range(_N_AGENTS):
          eidx_v[c * _NCHUNK + j, pl.ds(g * _L, _L)] = addr0 + c * _BN
        return carry

      lax.fori_loop(0, _GCHUNK // _L, jbody, 0)

    copies = [
        pltpu.async_copy(w_hbm.at[eidx_v.at[n]], rows_v.at[n], sem)
        for n in range(_N_AGENTS * _NCHUNK)
    ]
    for cp in copies:
      cp.wait()
    for c in range(_N_AGENTS):
      for j in range(_NCHUNK):
        pltpu.sync_copy(
            rows_v.at[c * _NCHUNK + j],
            o_cols[c].at[pl.ds(base + j * _GCHUNK, _GCHUNK)],
        )

  return _k


_sc_gather = _build_gather()


def kernel(a_joint, U):
  a_cols = tuple(a_joint[:, i] for i in range(_N_AGENTS))
  w_flat = _tc_pack(U.T).reshape(_WWORDS)
  outs = _sc_gather(*a_cols, w_flat)
  return jnp.stack(outs, axis=1)


# speedup vs baseline: 10.9008x; 1.0013x over previous
import functools

import jax
import jax.numpy as jnp
from jax import lax
from jax.experimental import pallas as pl
from jax.experimental.pallas import tpu as pltpu
from jax.experimental.pallas import tpu_sc as plsc

_N_AGENTS = 6
_N_ACTIONS = 10
_BATCH = 16384
_K = 1000000
_NC = 2
_NS = 16
_L = 16
_NW = _NC * _NS
_BPW = _BATCH // _NW
_GCHUNK = 128
_NCHUNK = _BPW // _GCHUNK

_BN = 131072
_NB = -(-_K // _BN)
_NQ = 2
_BSHIFT = _BN.bit_length() - 1
_WWORDS = _NB * _N_AGENTS * _BN


def _pack_body(ut0_ref, ut1_ref, w_ref):
  w_ref[0] = ut0_ref[...].reshape(_N_AGENTS, _BN // 128, 128)
  w_ref[1] = ut1_ref[...].reshape(_N_AGENTS, _BN // 128, 128)


def _build_pack():
  return pl.pallas_call(
      _pack_body,
      grid=(_NB // _NQ,),
      in_specs=[
          pl.BlockSpec((_N_AGENTS, _BN), lambda k: (0, _NQ * k)),
          pl.BlockSpec((_N_AGENTS, _BN), lambda k: (0, _NQ * k + 1)),
      ],
      out_specs=pl.BlockSpec(
          (_NQ, _N_AGENTS, _BN // 128, 128), lambda k: (k, 0, 0, 0)),
      out_shape=jax.ShapeDtypeStruct(
          (_NB, _N_AGENTS, _BN // 128, 128), jnp.float32),
  )


_tc_pack = _build_pack()


def _build_gather():
  mesh = plsc.VectorSubcoreMesh(core_axis_name="c", subcore_axis_name="s")

  @functools.partial(
      pl.kernel,
      mesh=mesh,
      out_type=tuple(
          jax.ShapeDtypeStruct((_BATCH,), jnp.float32)
          for _ in range(_N_AGENTS)
      ),
      compiler_params=pltpu.CompilerParams(
          use_tc_tiling_on_sc=False, needs_layout_passes=False),
      scratch_types=[
          pltpu.VMEM((_N_AGENTS, _BPW), jnp.int32),
          pltpu.VMEM((_N_AGENTS * _NCHUNK, _GCHUNK), jnp.int32),
          pltpu.VMEM((_N_AGENTS * _NCHUNK, _GCHUNK), jnp.float32),
          pltpu.SemaphoreType.DMA,
      ],
  )
  def _k(a0, a1, a2, a3, a4, a5, w_hbm,
         o0, o1, o2, o3, o4, o5, a_v, eidx_v, rows_v, sem):
    a_cols = (a0, a1, a2, a3, a4, a5)
    o_cols = (o0, o1, o2, o3, o4, o5)
    wid = lax.axis_index("s") * _NC + lax.axis_index("c")
    base = wid * _BPW
    for i in range(_N_AGENTS):
      pltpu.sync_copy(a_cols[i].at[pl.ds(base, _BPW)], a_v.at[i])

    for j in range(_NCHUNK):
      def jbody(g, carry, j=j):
        off = j * _GCHUNK + g * _L
        acc = jnp.zeros((_L,), jnp.int32)
        scale = 1
        for i in range(_N_AGENTS):
          acc = acc + a_v[i, pl.ds(off, _L)] * scale
          scale *= _N_ACTIONS
        addr0 = (acc >> _BSHIFT) * (_N_AGENTS * _BN) + (acc & (_BN - 1))
        for c in range(_N_AGENTS):
          eidx_v[c * _NCHUNK + j, pl.ds(g * _L, _L)] = addr0 + c * _BN
        return carry

      lax.fori_loop(0, _GCHUNK // _L, jbody, 0)

    copies = [
        pltpu.async_copy(w_hbm.at[eidx_v.at[n]], rows_v.at[n], sem)
        for n in range(_N_AGENTS * _NCHUNK)
    ]
    for cp in copies:
      cp.wait()
    for c in range(_N_AGENTS):
      for j in range(_NCHUNK):
        pltpu.sync_copy(
            rows_v.at[c * _NCHUNK + j],
            o_cols[c].at[pl.ds(base + j * _GCHUNK, _GCHUNK)],
        )

  return _k


_sc_gather = _build_gather()


def kernel(a_joint, U):
  a_cols = tuple(a_joint[:, i] for i in range(_N_AGENTS))
  ut = U.T
  w_flat = _tc_pack(ut, ut).reshape(_WWORDS)
  outs = _sc_gather(*a_cols, w_flat)
  return jnp.stack(outs, axis=1)


# speedup vs baseline: 10.9289x; 1.0026x over previous
import functools

import jax
import jax.numpy as jnp
from jax import lax
from jax.experimental import pallas as pl
from jax.experimental.pallas import tpu as pltpu
from jax.experimental.pallas import tpu_sc as plsc

_N_AGENTS = 6
_N_ACTIONS = 10
_BATCH = 16384
_K = 1000000
_NC = 2
_NS = 16
_L = 16
_NW = _NC * _NS
_BPW = _BATCH // _NW
_GCHUNK = 128
_NCHUNK = _BPW // _GCHUNK

_BN = 262144
_NB = -(-_K // _BN)
_BSHIFT = _BN.bit_length() - 1
_WWORDS = _NB * _N_AGENTS * _BN


def _pack_body(ut_ref, w_ref):
  w_ref[...] = ut_ref[...].reshape(1, _N_AGENTS, _BN // 128, 128)


def _build_pack():
  return pl.pallas_call(
      _pack_body,
      grid=(_NB,),
      in_specs=[pl.BlockSpec((_N_AGENTS, _BN), lambda k: (0, k))],
      out_specs=pl.BlockSpec(
          (1, _N_AGENTS, _BN // 128, 128), lambda k: (k, 0, 0, 0)),
      out_shape=jax.ShapeDtypeStruct(
          (_NB, _N_AGENTS, _BN // 128, 128), jnp.float32),
  )


_tc_pack = _build_pack()


def _build_gather():
  mesh = plsc.VectorSubcoreMesh(core_axis_name="c", subcore_axis_name="s")

  @functools.partial(
      pl.kernel,
      mesh=mesh,
      out_type=tuple(
          jax.ShapeDtypeStruct((_BATCH,), jnp.float32)
          for _ in range(_N_AGENTS)
      ),
      compiler_params=pltpu.CompilerParams(
          use_tc_tiling_on_sc=False, needs_layout_passes=False),
      scratch_types=[
          pltpu.VMEM((_N_AGENTS, _BPW), jnp.int32),
          pltpu.VMEM((_N_AGENTS * _NCHUNK, _GCHUNK), jnp.int32),
          pltpu.VMEM((_N_AGENTS * _NCHUNK, _GCHUNK), jnp.float32),
          pltpu.SemaphoreType.DMA,
      ],
  )
  def _k(a0, a1, a2, a3, a4, a5, w_hbm,
         o0, o1, o2, o3, o4, o5, a_v, eidx_v, rows_v, sem):
    a_cols = (a0, a1, a2, a3, a4, a5)
    o_cols = (o0, o1, o2, o3, o4, o5)
    wid = lax.axis_index("s") * _NC + lax.axis_index("c")
    base = wid * _BPW
    for i in range(_N_AGENTS):
      pltpu.sync_copy(a_cols[i].at[pl.ds(base, _BPW)], a_v.at[i])

    for j in range(_NCHUNK):
      def jbody(g, carry, j=j):
        off = j * _GCHUNK + g * _L
        acc = jnp.zeros((_L,), jnp.int32)
        scale = 1
        for i in range(_N_AGENTS):
          acc = acc + a_v[i, pl.ds(off, _L)] * scale
          scale *= _N_ACTIONS
        addr0 = (acc >> _BSHIFT) * (_N_AGENTS * _BN) + (acc & (_BN - 1))
        for c in range(_N_AGENTS):
          eidx_v[c * _NCHUNK + j, pl.ds(g * _L, _L)] = addr0 + c * _BN
        return carry

      lax.fori_loop(0, _GCHUNK // _L, jbody, 0)

    copies = [
        pltpu.async_copy(w_hbm.at[eidx_v.at[n]], rows_v.at[n], sem)
        for n in range(_N_AGENTS * _NCHUNK)
    ]
    for cp in copies:
      cp.wait()
    for c in range(_N_AGENTS):
      for j in range(_NCHUNK):
        pltpu.sync_copy(
            rows_v.at[c * _NCHUNK + j],
            o_cols[c].at[pl.ds(base + j * _GCHUNK, _GCHUNK)],
        )

  return _k


_sc_gather = _build_gather()


def kernel(a_joint, U):
  a_cols = tuple(a_joint[:, i] for i in range(_N_AGENTS))
  w_flat = _tc_pack(U.T).reshape(_WWORDS)
  outs = _sc_gather(*a_cols, w_flat)
  return jnp.stack(outs, axis=1)
